# Initial kernel scaffold; baseline (speedup 1.0000x reference)
import functools

import jax
import jax.numpy as jnp
from jax import lax
from jax.experimental import pallas as pl
from jax.experimental.pallas import tpu as pltpu
from jax.experimental.pallas import tpu_sc as plsc

NC = 2
NS = 16
LANES = 16
CHUNK = 16


def _build_spmm(B, N, D, EP):
    assert EP % (NC * NS * CHUNK) == 0
    assert N % NS == 0
    EPW = EP // (NC * NS)
    STRIPE = N // NS
    ZROWS = 125
    assert STRIPE % ZROWS == 0
    mesh = plsc.VectorSubcoreMesh(core_axis_name="c", subcore_axis_name="s")

    @functools.partial(
        pl.kernel,
        out_type=jax.ShapeDtypeStruct((NC, B * N, D), jnp.float32),
        mesh=mesh,
        scratch_types=[
            pltpu.VMEM((EPW,), jnp.int32),
            pltpu.VMEM((EPW,), jnp.int32),
            pltpu.VMEM((EPW,), jnp.float32),
            pltpu.VMEM((CHUNK, D), jnp.float32),
            pltpu.VMEM((125, D), jnp.float32),
            pltpu.VMEM_SHARED((N, D), jnp.float32),
        ],
    )
    def spmm(h_hbm, col_hbm, row_hbm, w_hbm, out_hbm,
             col_v, row_v, w_v, gbuf, zbuf, acc):
        c = lax.axis_index("c")
        s = lax.axis_index("s")
        wid = s * NC + c
        base = wid * EPW
        pltpu.sync_copy(col_hbm.at[pl.ds(base, EPW)], col_v)
        pltpu.sync_copy(row_hbm.at[pl.ds(base, EPW)], row_v)
        pltpu.sync_copy(w_hbm.at[pl.ds(base, EPW)], w_v)

        ZROWS = 125

        def zinit(r, _):
            for j in range(D // LANES):
                zbuf[r, pl.ds(j * LANES, LANES)] = jnp.zeros((LANES,), jnp.float32)
            return ()
        lax.fori_loop(0, ZROWS, zinit, ())

        for b in range(B):
            for k in range(STRIPE // ZROWS):
                pltpu.sync_copy(zbuf, acc.at[pl.ds(s * STRIPE + k * ZROWS, ZROWS)])
            plsc.subcore_barrier()

            def chunk_body(i, _):
                e0 = i * CHUNK
                cv = col_v[pl.ds(e0, CHUNK)] + b * N
                pltpu.sync_copy(h_hbm.at[cv], gbuf)
                for e in range(CHUNK):
                    we = plsc.load_gather(
                        w_v, [jnp.zeros((LANES,), jnp.int32) + (e0 + e)])
                    for j in range(D // LANES):
                        gbuf[e, pl.ds(j * LANES, LANES)] = (
                            gbuf[e, pl.ds(j * LANES, LANES)] * we)
                rv = row_v[pl.ds(e0, CHUNK)]
                pltpu.sync_copy(gbuf, acc.at[rv], add=True)
                return ()
            lax.fori_loop(0, EPW // CHUNK, chunk_body, ())
            plsc.subcore_barrier()
            pltpu.sync_copy(
                acc.at[pl.ds(s * STRIPE, STRIPE)],
                out_hbm.at[c, pl.ds(b * N + s * STRIPE, STRIPE)])
            plsc.subcore_barrier()

    return spmm


def _tc_layer(partials, W, bvec, res, final):
    BN, D = partials.shape[1], partials.shape[2]
    R = 1000
    assert BN % R == 0
    grid = (BN // R,)

    def body(p_ref, w_ref, b_ref, r_ref, o_ref):
        agg = p_ref[0] + p_ref[1]
        z = jnp.dot(agg, w_ref[...], preferred_element_type=jnp.float32)
        z = z + b_ref[...]
        if final:
            o_ref[...] = jax.nn.sigmoid(jax.nn.relu(z))
        else:
            o_ref[...] = jax.nn.relu(z + r_ref[...])

    return pl.pallas_call(
        body,
        grid=grid,
        in_specs=[
            pl.BlockSpec((NC, R, D), lambda i: (0, i, 0)),
            pl.BlockSpec((D, D), lambda i: (0, 0)),
            pl.BlockSpec((1, D), lambda i: (0, 0)),
            pl.BlockSpec((R, D), lambda i: (i, 0)),
        ],
        out_specs=pl.BlockSpec((R, D), lambda i: (i, 0)),
        out_shape=jax.ShapeDtypeStruct((BN, D), jnp.float32),
    )(partials, W, bvec.reshape(1, D), res)


def kernel(x, edge_index, edge_weight, W1, b1, W2, b2, W3, b3):
    B, N, D = x.shape
    E = edge_weight.shape[0]
    quantum = NC * NS * CHUNK
    EP = ((E + quantum - 1) // quantum) * quantum
    col = edge_index[1]
    row = edge_index[0]
    w = edge_weight
    if EP != E:
        pad = EP - E
        col = jnp.concatenate([col, jnp.zeros((pad,), jnp.int32)])
        row = jnp.concatenate([row, jnp.zeros((pad,), jnp.int32)])
        w = jnp.concatenate([w, jnp.zeros((pad,), jnp.float32)])

    spmm = _build_spmm(B, N, D, EP)
    H = x.reshape(B * N, D)
    for Wl, bl, final in ((W1, b1, False), (W2, b2, False), (W3, b3, True)):
        partials = spmm(H, col, row, w)
        H = _tc_layer(partials, Wl, bl, H, final)
    return H.reshape(B, N, D)


# speedup vs baseline: 2.1418x; 2.1418x over previous
import functools

import jax
import jax.numpy as jnp
from jax import lax
from jax.experimental import pallas as pl
from jax.experimental.pallas import tpu as pltpu
from jax.experimental.pallas import tpu_sc as plsc

NC = 2
NS = 16
LANES = 16
CHUNK = 16


def _build_spmm(B, N, D, EP):
    assert EP % (NC * NS * CHUNK) == 0
    assert N % (NS * 8) == 0
    EPW = EP // (NC * NS)
    STRIPE = N // NS
    ZROWS = 128
    assert STRIPE % ZROWS == 0
    mesh = plsc.VectorSubcoreMesh(core_axis_name="c", subcore_axis_name="s")

    @functools.partial(
        pl.kernel,
        out_type=jax.ShapeDtypeStruct((NC, B * N, D), jnp.float32),
        mesh=mesh,
        scratch_types=[
            pltpu.VMEM((EPW,), jnp.int32),
            pltpu.VMEM((EPW,), jnp.int32),
            pltpu.VMEM((EPW,), jnp.float32),
            pltpu.VMEM((CHUNK, D), jnp.float32),
            pltpu.VMEM((128, D), jnp.float32),
            pltpu.VMEM_SHARED((N, D), jnp.float32),
        ],
    )
    def spmm(h_hbm, col_hbm, row_hbm, w_hbm, out_hbm,
             col_v, row_v, w_v, gbuf, zbuf, acc):
        c = lax.axis_index("c")
        s = lax.axis_index("s")
        wid = s * NC + c
        base = wid * EPW
        pltpu.sync_copy(col_hbm.at[pl.ds(base, EPW)], col_v)
        pltpu.sync_copy(row_hbm.at[pl.ds(base, EPW)], row_v)
        pltpu.sync_copy(w_hbm.at[pl.ds(base, EPW)], w_v)

        def zinit(r, _):
            for j in range(D // LANES):
                zbuf[r, pl.ds(j * LANES, LANES)] = jnp.zeros((LANES,), jnp.float32)
            return ()
        lax.fori_loop(0, ZROWS, zinit, ())

        for b in range(B):
            for k in range(STRIPE // ZROWS):
                pltpu.sync_copy(zbuf, acc.at[pl.ds(s * STRIPE + k * ZROWS, ZROWS)])
            plsc.subcore_barrier()

            dn = lax.GatherDimensionNumbers(
                offset_dims=(), collapsed_slice_dims=(0,), start_index_map=(0,))

            def chunk_body(i, _):
                e0 = i * CHUNK
                cv = col_v[pl.ds(e0, CHUNK)] + b * N
                pltpu.sync_copy(h_hbm.at[cv], gbuf)
                wv = w_v[pl.ds(e0, CHUNK)]
                for e in range(CHUNK):
                    idx = jnp.zeros((LANES,), jnp.int32) + e
                    we = lax.gather(wv, idx[:, None], dn, slice_sizes=(1,),
                                    mode=lax.GatherScatterMode.PROMISE_IN_BOUNDS)
                    for j in range(D // LANES):
                        gbuf[e, pl.ds(j * LANES, LANES)] = (
                            gbuf[e, pl.ds(j * LANES, LANES)] * we)
                rv = row_v[pl.ds(e0, CHUNK)]
                pltpu.sync_copy(gbuf, acc.at[rv], add=True)
                return ()
            lax.fori_loop(0, EPW // CHUNK, chunk_body, ())
            plsc.subcore_barrier()
            pltpu.sync_copy(
                acc.at[pl.ds(s * STRIPE, STRIPE)],
                out_hbm.at[c, pl.ds(b * N + s * STRIPE, STRIPE)])
            plsc.subcore_barrier()

    return spmm


def _tc_matmul(H, W):
    BN, D = H.shape
    R = 1024
    assert BN % R == 0

    def body(h_ref, w_ref, o_ref):
        o_ref[...] = jnp.dot(h_ref[...], w_ref[...],
                             preferred_element_type=jnp.float32)

    return pl.pallas_call(
        body,
        grid=(BN // R,),
        in_specs=[
            pl.BlockSpec((R, D), lambda i: (i, 0)),
            pl.BlockSpec((D, D), lambda i: (0, 0)),
        ],
        out_specs=pl.BlockSpec((R, D), lambda i: (i, 0)),
        out_shape=jax.ShapeDtypeStruct((BN, D), jnp.float32),
    )(H, W)


def _tc_epi_mm(partials, bvec, res, Wn):
    BN, D = partials.shape[1], partials.shape[2]
    R = 1024
    assert BN % R == 0

    def body(p_ref, b_ref, r_ref, w_ref, h_ref, g_ref):
        h = jax.nn.relu(p_ref[0] + p_ref[1] + b_ref[...] + r_ref[...])
        h_ref[...] = h
        g_ref[...] = jnp.dot(h, w_ref[...], preferred_element_type=jnp.float32)

    return pl.pallas_call(
        body,
        grid=(BN // R,),
        in_specs=[
            pl.BlockSpec((NC, R, D), lambda i: (0, i, 0)),
            pl.BlockSpec((1, D), lambda i: (0, 0)),
            pl.BlockSpec((R, D), lambda i: (i, 0)),
            pl.BlockSpec((D, D), lambda i: (0, 0)),
        ],
        out_specs=[
            pl.BlockSpec((R, D), lambda i: (i, 0)),
            pl.BlockSpec((R, D), lambda i: (i, 0)),
        ],
        out_shape=[
            jax.ShapeDtypeStruct((BN, D), jnp.float32),
            jax.ShapeDtypeStruct((BN, D), jnp.float32),
        ],
    )(partials, bvec.reshape(1, D), res, Wn)


def _tc_epi_final(partials, bvec):
    BN, D = partials.shape[1], partials.shape[2]
    R = 1024
    assert BN % R == 0

    def body(p_ref, b_ref, o_ref):
        o_ref[...] = jax.nn.sigmoid(
            jax.nn.relu(p_ref[0] + p_ref[1] + b_ref[...]))

    return pl.pallas_call(
        body,
        grid=(BN // R,),
        in_specs=[
            pl.BlockSpec((NC, R, D), lambda i: (0, i, 0)),
            pl.BlockSpec((1, D), lambda i: (0, 0)),
        ],
        out_specs=pl.BlockSpec((R, D), lambda i: (i, 0)),
        out_shape=jax.ShapeDtypeStruct((BN, D), jnp.float32),
    )(partials, bvec.reshape(1, D))


def kernel(x, edge_index, edge_weight, W1, b1, W2, b2, W3, b3):
    B, N, D = x.shape
    E = edge_weight.shape[0]
    quantum = NC * NS * CHUNK
    EP = ((E + quantum - 1) // quantum) * quantum
    col = edge_index[1]
    row = edge_index[0]
    w = edge_weight
    if EP != E:
        pad = EP - E
        col = jnp.concatenate([col, jnp.zeros((pad,), jnp.int32)])
        row = jnp.concatenate([row, jnp.zeros((pad,), jnp.int32)])
        w = jnp.concatenate([w, jnp.zeros((pad,), jnp.float32)])

    N2 = ((N + NS * 128 - 1) // (NS * 128)) * (NS * 128)
    Hp = jnp.zeros((B, N2, D), jnp.float32).at[:, :N, :].set(x)

    spmm = _build_spmm(B, N2, D, EP)
    H = Hp.reshape(B * N2, D)
    G = _tc_matmul(H, W1)
    p = spmm(G, col, row, w)
    H, G = _tc_epi_mm(p, b1, H, W2)
    p = spmm(G, col, row, w)
    H, G = _tc_epi_mm(p, b2, H, W3)
    p = spmm(G, col, row, w)
    out = _tc_epi_final(p, b3)
    return out.reshape(B, N2, D)[:, :N, :]


# speedup vs baseline: 3.2015x; 1.4948x over previous
import functools

import jax
import jax.numpy as jnp
from jax import lax
from jax.experimental import pallas as pl
from jax.experimental.pallas import tpu as pltpu
from jax.experimental.pallas import tpu_sc as plsc

NC = 2
NS = 16
LANES = 16
CHUNK = 64


def _build_spmm(B, N, D, NCH):
    NW = NC * NS
    EPW = NCH * CHUNK
    G16 = CHUNK // LANES
    assert NCH % 2 == 0
    STRIPE = N // NS
    mesh = plsc.VectorSubcoreMesh(core_axis_name="c", subcore_axis_name="s")

    @functools.partial(
        pl.kernel,
        out_type=jax.ShapeDtypeStruct((NC, B * N, D), jnp.float32),
        mesh=mesh,
        scratch_types=[
            pltpu.VMEM((EPW + 2 * CHUNK,), jnp.int32),
            pltpu.VMEM((EPW,), jnp.int32),
            pltpu.VMEM((EPW,), jnp.float32),
            pltpu.VMEM((CHUNK, D), jnp.float32),
            pltpu.VMEM((CHUNK, D), jnp.float32),
            pltpu.VMEM_SHARED((N, D), jnp.float32),
            pltpu.SemaphoreType.DMA,
            pltpu.SemaphoreType.DMA,
            pltpu.SemaphoreType.DMA,
            pltpu.SemaphoreType.DMA,
        ],
    )
    def spmm(h_hbm, col_hbm, row_hbm, w_hbm, zeros_hbm, out_hbm,
             colv, rowv, wv, g0, g1, acc, gs0, gs1, ss0, ss1):
        c = lax.axis_index("c")
        s = lax.axis_index("s")
        wid = s * NC + c
        pltpu.sync_copy(row_hbm.at[wid], rowv)
        pltpu.sync_copy(w_hbm.at[wid], wv)

        dn = lax.GatherDimensionNumbers(
            offset_dims=(), collapsed_slice_dims=(0,), start_index_map=(0,))

        def gather_chunk(j, gb, gs):
            for g in range(G16):
                cv = colv[pl.ds(j * CHUNK + g * LANES, LANES)]
                pltpu.async_copy(h_hbm.at[cv], gb.at[pl.ds(g * LANES, LANES)], gs)

        def drain(buf, sem):
            pltpu.make_async_copy(h_hbm.at[pl.ds(0, CHUNK)], buf, sem).wait()

        def batch_body(b, _):
            pltpu.sync_copy(col_hbm.at[b * NW + wid], colv)
            pltpu.sync_copy(zeros_hbm, acc.at[pl.ds(s * STRIPE, STRIPE)])
            plsc.subcore_barrier()

            gather_chunk(0, g0, gs0)
            gather_chunk(1, g1, gs1)

            def pair(i2, _):
                for k in range(2):
                    jj = i2 * 2 + k
                    gb, gs, ss = (g0, gs0, ss0) if k == 0 else (g1, gs1, ss1)
                    drain(gb, gs)
                    for g in range(G16):
                        wvg = wv[pl.ds(jj * CHUNK + g * LANES, LANES)]
                        for e16 in range(LANES):
                            idx = jnp.zeros((LANES,), jnp.int32) + e16
                            we = lax.gather(
                                wvg, idx[:, None], dn, slice_sizes=(1,),
                                mode=lax.GatherScatterMode.PROMISE_IN_BOUNDS)
                            eg = g * LANES + e16
                            for j in range(D // LANES):
                                gb[eg, pl.ds(j * LANES, LANES)] = (
                                    gb[eg, pl.ds(j * LANES, LANES)] * we)
                    for g in range(G16):
                        rv = rowv[pl.ds(jj * CHUNK + g * LANES, LANES)]
                        pltpu.async_copy(gb.at[pl.ds(g * LANES, LANES)],
                                         acc.at[rv], ss, add=True)
                    drain(gb, ss)
                    gather_chunk(jj + 2, gb, gs)
                return ()
            lax.fori_loop(0, NCH // 2, pair, ())
            drain(g0, gs0)
            drain(g1, gs1)
            plsc.subcore_barrier()
            pltpu.sync_copy(
                acc.at[pl.ds(s * STRIPE, STRIPE)],
                out_hbm.at[c, pl.ds(b * N + s * STRIPE, STRIPE)])
            plsc.subcore_barrier()
            return ()
        lax.fori_loop(0, B, batch_body, ())

    return spmm


def _tc_matmul(H, W):
    BN, D = H.shape
    R = 1024
    assert BN % R == 0

    def body(h_ref, w_ref, o_ref):
        o_ref[...] = jnp.dot(h_ref[...], w_ref[...],
                             preferred_element_type=jnp.float32)

    return pl.pallas_call(
        body,
        grid=(BN // R,),
        in_specs=[
            pl.BlockSpec((R, D), lambda i: (i, 0)),
            pl.BlockSpec((D, D), lambda i: (0, 0)),
        ],
        out_specs=pl.BlockSpec((R, D), lambda i: (i, 0)),
        out_shape=jax.ShapeDtypeStruct((BN, D), jnp.float32),
    )(H, W)


def _tc_epi_mm(partials, bvec, res, Wn):
    BN, D = partials.shape[1], partials.shape[2]
    R = 1024
    assert BN % R == 0

    def body(p_ref, b_ref, r_ref, w_ref, h_ref, g_ref):
        h = jax.nn.relu(p_ref[0] + p_ref[1] + b_ref[...] + r_ref[...])
        h_ref[...] = h
        g_ref[...] = jnp.dot(h, w_ref[...], preferred_element_type=jnp.float32)

    return pl.pallas_call(
        body,
        grid=(BN // R,),
        in_specs=[
            pl.BlockSpec((NC, R, D), lambda i: (0, i, 0)),
            pl.BlockSpec((1, D), lambda i: (0, 0)),
            pl.BlockSpec((R, D), lambda i: (i, 0)),
            pl.BlockSpec((D, D), lambda i: (0, 0)),
        ],
        out_specs=[
            pl.BlockSpec((R, D), lambda i: (i, 0)),
            pl.BlockSpec((R, D), lambda i: (i, 0)),
        ],
        out_shape=[
            jax.ShapeDtypeStruct((BN, D), jnp.float32),
            jax.ShapeDtypeStruct((BN, D), jnp.float32),
        ],
    )(partials, bvec.reshape(1, D), res, Wn)


def _tc_epi_final(partials, bvec):
    BN, D = partials.shape[1], partials.shape[2]
    R = 1024
    assert BN % R == 0

    def body(p_ref, b_ref, o_ref):
        o_ref[...] = jax.nn.sigmoid(
            jax.nn.relu(p_ref[0] + p_ref[1] + b_ref[...]))

    return pl.pallas_call(
        body,
        grid=(BN // R,),
        in_specs=[
            pl.BlockSpec((NC, R, D), lambda i: (0, i, 0)),
            pl.BlockSpec((1, D), lambda i: (0, 0)),
        ],
        out_specs=pl.BlockSpec((R, D), lambda i: (i, 0)),
        out_shape=jax.ShapeDtypeStruct((BN, D), jnp.float32),
    )(partials, bvec.reshape(1, D))


def kernel(x, edge_index, edge_weight, W1, b1, W2, b2, W3, b3):
    B, N, D = x.shape
    E = edge_weight.shape[0]
    NW = NC * NS
    quantum = NW * CHUNK * 2
    EP = ((E + quantum - 1) // quantum) * quantum
    EPW = EP // NW
    NCH = EPW // CHUNK
    col = edge_index[1]
    row = edge_index[0]
    w = edge_weight
    if EP != E:
        pad = EP - E
        col = jnp.concatenate([col, jnp.zeros((pad,), jnp.int32)])
        row = jnp.concatenate([row, jnp.zeros((pad,), jnp.int32)])
        w = jnp.concatenate([w, jnp.zeros((pad,), jnp.float32)])

    N2 = ((N + NS * 128 - 1) // (NS * 128)) * (NS * 128)
    Hp = jnp.zeros((B, N2, D), jnp.float32).at[:, :N, :].set(x)

    colw = col.reshape(NW, EPW)
    col4 = colw[None] + (jnp.arange(B, dtype=jnp.int32) * N2)[:, None, None]
    col4 = col4.reshape(B * NW, EPW)
    col4 = jnp.concatenate(
        [col4, jnp.zeros((B * NW, 2 * CHUNK), jnp.int32)], axis=1)
    row2 = row.reshape(NW, EPW)
    w2 = w.reshape(NW, EPW)

    spmm = _build_spmm(B, N2, D, NCH)
    zeros = jnp.zeros((N2 // NS, D), jnp.float32)
    H = Hp.reshape(B * N2, D)
    G = _tc_matmul(H, W1)
    p = spmm(G, col4, row2, w2, zeros)
    H, G = _tc_epi_mm(p, b1, H, W2)
    p = spmm(G, col4, row2, w2, zeros)
    H, G = _tc_epi_mm(p, b2, H, W3)
    p = spmm(G, col4, row2, w2, zeros)
    out = _tc_epi_final(p, b3)
    return out.reshape(B, N2, D)[:, :N, :]
